# baseline (speedup 1.0000x reference)
import functools

import jax
import jax.numpy as jnp
from jax import lax
from jax.experimental import pallas as pl
from jax.experimental.pallas import tpu as pltpu
from jax.experimental.pallas import tpu_sc as plsc

_N = 10000
_E = 320000
_D = 128
_G = 5
_GP = 8
_LAM = 0.01
_EPS = 1e-5

_NC = 2
_NS = 16
_NW = _NC * _NS
_EPW = _E // _NW
_CH = 128
_NCHUNK = -(-_EPW // _CH)
_EPAD = _NCHUNK * _CH
_NP = 10240
_RPT = _NP // _NS

_MESH = dict(core_axis_name="c", subcore_axis_name="s", num_cores=_NC,
             num_subcores=_NS)


def _sc_deg_body(comb_hbm, out_hbm, idx_v, dbuf, ones_v, z_v, hist_sh):
    c = lax.axis_index("c")
    s = lax.axis_index("s")
    w = c * _NS + s

    def fill_ones(i, carry):
        ones_v[pl.ds(i * 16, 16)] = jnp.full((16,), 1.0, jnp.float32)
        return carry

    lax.fori_loop(0, _CH // 16, fill_ones, 0)

    def fill_z(i, carry):
        z_v[pl.ds(i * 16, 16)] = jnp.zeros((16,), jnp.float32)
        return carry

    lax.fori_loop(0, _RPT // 16, fill_z, 0)

    pltpu.sync_copy(comb_hbm.at[w], idx_v)
    pltpu.sync_copy(z_v, hist_sh.at[pl.ds(s * _RPT, _RPT)])
    plsc.subcore_barrier()

    def chunk(j, carry):
        def unpack(k, carry2):
            dbuf[0, pl.ds(k * 16, 16)] = lax.shift_right_logical(
                idx_v[j, pl.ds(k * 16, 16)], 16)
            return carry2

        lax.fori_loop(0, _CH // 16, unpack, 0)
        pltpu.sync_copy(ones_v, hist_sh.at[dbuf.at[0]], add=True)
        return carry

    lax.fori_loop(0, _NCHUNK, chunk, 0)
    plsc.subcore_barrier()
    pltpu.sync_copy(hist_sh.at[pl.ds(s * _RPT, _RPT)], out_hbm.at[c, s])


def _sc_scatter_body(hs_hbm, comb_hbm, out_hbm, idx_v, sbuf, dbuf, gbuf,
                     acc_sh, gsem, ssem):
    c = lax.axis_index("c")
    s = lax.axis_index("s")
    w = c * _NS + s

    def zrow(r, carry):
        def zcol(k, carry2):
            gbuf[0, r, pl.ds(k * 16, 16)] = jnp.zeros((16,), jnp.float32)
            return carry2

        lax.fori_loop(0, _D // 16, zcol, 0)
        return carry

    lax.fori_loop(0, _CH, zrow, 0)

    def zacc(k, carry):
        pltpu.sync_copy(gbuf.at[0, pl.ds(0, 64)],
                        acc_sh.at[pl.ds(s * _RPT + k * 64, 64)])
        return carry

    lax.fori_loop(0, _RPT // 64, zacc, 0)

    pltpu.sync_copy(comb_hbm.at[w], idx_v)
    plsc.subcore_barrier()

    def unpack(j):
        m = lax.rem(j, 2)

        def body(k, carry):
            v = idx_v[j, pl.ds(k * 16, 16)]
            sbuf[m, pl.ds(k * 16, 16)] = lax.bitwise_and(
                v, jnp.full((16,), 0xFFFF, jnp.int32))
            dbuf[m, pl.ds(k * 16, 16)] = lax.shift_right_logical(v, 16)
            return carry

        lax.fori_loop(0, _CH // 16, body, 0)

    def start_gather(j):
        m = lax.rem(j, 2)
        pltpu.async_copy(hs_hbm.at[sbuf.at[m]], gbuf.at[m], gsem.at[m])

    def wait_gather(j):
        m = lax.rem(j, 2)
        pltpu.make_async_copy(hs_hbm.at[sbuf.at[m]], gbuf.at[m],
                              gsem.at[m]).wait()

    def start_scatter(j):
        m = lax.rem(j, 2)
        pltpu.async_copy(gbuf.at[m], acc_sh.at[dbuf.at[m]], ssem.at[m],
                         add=True)

    def wait_scatter(j):
        m = lax.rem(j, 2)
        pltpu.make_async_copy(gbuf.at[m], acc_sh.at[dbuf.at[m]],
                              ssem.at[m]).wait()

    unpack(0)
    start_gather(0)
    unpack(1)
    start_gather(1)
    wait_gather(0)
    start_scatter(0)

    def chunk(j, carry):
        wait_scatter(j - 1)
        unpack(j + 1)
        start_gather(j + 1)
        wait_gather(j)
        start_scatter(j)
        return carry

    lax.fori_loop(1, _NCHUNK - 1, chunk, 0)
    wait_gather(_NCHUNK - 1)
    start_scatter(_NCHUNK - 1)
    wait_scatter(_NCHUNK - 2)
    wait_scatter(_NCHUNK - 1)
    plsc.subcore_barrier()
    pltpu.sync_copy(acc_sh.at[pl.ds(s * _RPT, _RPT)], out_hbm.at[c, s])


@functools.cache
def _sc_kernels():
    mesh = plsc.VectorSubcoreMesh(**_MESH)
    sc_deg = pl.kernel(
        _sc_deg_body,
        out_type=jax.ShapeDtypeStruct((_NC, _NS, _RPT), jnp.float32),
        mesh=mesh,
        scratch_types=[
            pltpu.VMEM((_NCHUNK, _CH), jnp.int32),
            pltpu.VMEM((1, _CH), jnp.int32),
            pltpu.VMEM((_CH,), jnp.float32),
            pltpu.VMEM((_RPT,), jnp.float32),
            pltpu.VMEM_SHARED((_NP,), jnp.float32),
        ],
    )
    sc_scatter = pl.kernel(
        _sc_scatter_body,
        out_type=jax.ShapeDtypeStruct((_NC, _NS, _RPT, _D), jnp.float32),
        mesh=mesh,
        scratch_types=[
            pltpu.VMEM((_NCHUNK, _CH), jnp.int32),
            pltpu.VMEM((2, _CH), jnp.int32),
            pltpu.VMEM((2, _CH), jnp.int32),
            pltpu.VMEM((2, _CH, _D), jnp.float32),
            pltpu.VMEM_SHARED((_NP, _D), jnp.float32),
            pltpu.SemaphoreType.DMA((2,)),
            pltpu.SemaphoreType.DMA((2,)),
        ],
    )
    return sc_deg, sc_scatter


def _dgn_relu(o, lw, lb, bw, bb):
    logits = jnp.dot(o, lw, preferred_element_type=jnp.float32) + lb[None, :]
    s = jax.nn.softmax(logits, axis=-1)
    cdims = (((0,), (0,)), ((), ()))
    m = lax.dot_general(s, o, cdims,
                        preferred_element_type=jnp.float32) / _N
    q = lax.dot_general(s * s, o * o, cdims,
                        preferred_element_type=jnp.float32) / _N
    var = q - m * m
    wgt = bw * lax.rsqrt(var + _EPS)
    cvec = jnp.sum(m * wgt - bb, axis=0, keepdims=True)
    sw = jnp.dot(s, wgt, preferred_element_type=jnp.float32)
    return jnp.maximum(o + _LAM * (o * sw - cvec), 0.0)


def _tc_prea_body(x_ref, wpre_ref, bpre_ref, w1_ref, hw_ref):
    h = jnp.dot(x_ref[...], wpre_ref[...],
                preferred_element_type=jnp.float32) + bpre_ref[...][None, :]
    hw_ref[...] = jnp.dot(h, w1_ref[...], preferred_element_type=jnp.float32)


def _tc_preb_body(hw_ref, degp_ref, hs_ref, dinv_ref):
    deg = degp_ref[0] + degp_ref[1]
    dinv = lax.rsqrt(deg[: _N] + 1.0)
    dinv_ref[...] = dinv
    hs_ref[...] = dinv * hw_ref[...]


def _tc_mid_body(acc_ref, hs1_ref, dinv_ref, b1_ref, lw_ref, lb_ref, bw_ref,
                 bb_ref, w2_ref, hs2_ref):
    accsum = acc_ref[0, : _N] + acc_ref[1, : _N]
    dinv = dinv_ref[...]
    o = dinv * (accsum + hs1_ref[...]) + b1_ref[...][None, :]
    h = _dgn_relu(o, lw_ref[...], lb_ref[...], bw_ref[...], bb_ref[...])
    hs2_ref[...] = dinv * jnp.dot(h, w2_ref[...],
                                  preferred_element_type=jnp.float32)


def _tc_post_body(acc_ref, hs2_ref, dinv_ref, b2_ref, lw_ref, lb_ref, bw_ref,
                  bb_ref, wjk_ref, bjk_ref, wpost_ref, bpost_ref, out_ref):
    accsum = acc_ref[0, : _N] + acc_ref[1, : _N]
    dinv = dinv_ref[...]
    o = dinv * (accsum + hs2_ref[...]) + b2_ref[...][None, :]
    h = _dgn_relu(o, lw_ref[...], lb_ref[...], bw_ref[...], bb_ref[...])
    t = jnp.dot(h, wjk_ref[...],
                preferred_element_type=jnp.float32) + bjk_ref[...][None, :]
    out_ref[...] = jnp.dot(t, wpost_ref[...],
                           preferred_element_type=jnp.float32) \
        + bpost_ref[...][None, :]


_tc_prea = pl.pallas_call(
    _tc_prea_body,
    out_shape=jax.ShapeDtypeStruct((_N, _D), jnp.float32),
)

_tc_preb = pl.pallas_call(
    _tc_preb_body,
    out_shape=[jax.ShapeDtypeStruct((_N, _D), jnp.float32),
               jax.ShapeDtypeStruct((_N, 1), jnp.float32)],
)

_tc_mid = pl.pallas_call(
    _tc_mid_body,
    out_shape=jax.ShapeDtypeStruct((_N, _D), jnp.float32),
)

_tc_post = pl.pallas_call(
    _tc_post_body,
    out_shape=jax.ShapeDtypeStruct((_N, _D), jnp.float32),
)


def _pad_group_params(lw, lb, bw, bb):
    lw_p = jnp.pad(lw, ((0, 0), (0, _GP - _G)))
    lb_p = jnp.pad(lb, (0, _GP - _G), constant_values=-1e30)
    bw_p = jnp.pad(bw.reshape(_G, _D), ((0, _GP - _G), (0, 0)))
    bb_p = jnp.pad(bb.reshape(_G, _D), ((0, _GP - _G), (0, 0)))
    return lw_p, lb_p, bw_p, bb_p


def kernel(x, adj, W_pre, b_pre, W1, b1, dgn1_lw, dgn1_lb, dgn1_bw, dgn1_bb,
           W2, b2, dgn2_lw, dgn2_lb, dgn2_bw, dgn2_bb, W_jk, b_jk, W_post,
           b_post):
    src, dst = adj[0], adj[1]
    npad = _EPAD - _EPW
    pad_src = (jnp.arange(npad, dtype=jnp.int32) * 97) % _N
    pad_dst = _N + (jnp.arange(npad, dtype=jnp.int32) % (_NP - _N))
    comb = src + dst * 65536
    pad_comb = pad_src + pad_dst * 65536
    combp = jnp.concatenate(
        [comb.reshape(_NW, _EPW),
         jnp.broadcast_to(pad_comb, (_NW, npad))], axis=1
    ).reshape(_NW, _NCHUNK, _CH)

    lw1, lb1, bw1, bb1 = _pad_group_params(dgn1_lw, dgn1_lb, dgn1_bw, dgn1_bb)
    lw2, lb2, bw2, bb2 = _pad_group_params(dgn2_lw, dgn2_lb, dgn2_bw, dgn2_bb)

    sc_deg, sc_scatter = _sc_kernels()
    degp = sc_deg(combp).reshape(_NC, _NP, 1)
    hw1 = _tc_prea(x, W_pre, b_pre, W1)
    hs1, dinv = _tc_preb(hw1, degp)
    acc1 = sc_scatter(hs1, combp).reshape(_NC, _NP, _D)
    hs2 = _tc_mid(acc1, hs1, dinv, b1, lw1, lb1, bw1, bb1, W2)
    acc2 = sc_scatter(hs2, combp).reshape(_NC, _NP, _D)
    out = _tc_post(acc2, hs2, dinv, b2, lw2, lb2, bw2, bb2, W_jk, b_jk,
                   W_post, b_post)
    return out


# speedup vs baseline: 1.0005x; 1.0005x over previous
import functools

import jax
import jax.numpy as jnp
from jax import lax
from jax.experimental import pallas as pl
from jax.experimental.pallas import tpu as pltpu
from jax.experimental.pallas import tpu_sc as plsc

_N = 10000
_E = 320000
_D = 128
_G = 5
_GP = 8
_LAM = 0.01
_EPS = 1e-5

_NC = 2
_NS = 16
_NW = _NC * _NS
_EPW = _E // _NW
_CH = 128
_NCHUNK = -(-_EPW // _CH)
_NP = 10240
_RPT = _NP // _NS

_MESH = dict(core_axis_name="c", subcore_axis_name="s", num_cores=_NC,
             num_subcores=_NS)


def _sc_deg_body(comb_hbm, out_hbm, idx_v, dbuf, ones_v, z_v, hist_sh):
    c = lax.axis_index("c")
    s = lax.axis_index("s")
    w = c * _NS + s

    def fill_ones(i, carry):
        ones_v[pl.ds(i * 16, 16)] = jnp.full((16,), 1.0, jnp.float32)
        return carry

    lax.fori_loop(0, _CH // 16, fill_ones, 0)

    def fill_z(i, carry):
        z_v[pl.ds(i * 16, 16)] = jnp.zeros((16,), jnp.float32)
        return carry

    lax.fori_loop(0, _RPT // 16, fill_z, 0)

    pltpu.sync_copy(comb_hbm.at[pl.ds(w * _EPW, _EPW)], idx_v)
    pltpu.sync_copy(z_v, hist_sh.at[pl.ds(s * _RPT, _RPT)])
    plsc.subcore_barrier()

    def chunk(j, carry):
        def unpack(k, carry2):
            base = j * _CH + k * 16

            @pl.when(base < _EPW)
            def _():
                dbuf[0, pl.ds(k * 16, 16)] = lax.shift_right_logical(
                    idx_v[pl.ds(base, 16)], 16)

            @pl.when(base >= _EPW)
            def _():
                dbuf[0, pl.ds(k * 16, 16)] = (
                    _N + k * 16 + lax.iota(jnp.int32, 16))

            return carry2

        lax.fori_loop(0, _CH // 16, unpack, 0)
        pltpu.sync_copy(ones_v, hist_sh.at[dbuf.at[0]], add=True)
        return carry

    lax.fori_loop(0, _NCHUNK, chunk, 0)
    plsc.subcore_barrier()
    pltpu.sync_copy(hist_sh.at[pl.ds(s * _RPT, _RPT)], out_hbm.at[c, s])


def _sc_scatter_body(hs_hbm, comb_hbm, out_hbm, idx_v, sbuf, dbuf, gbuf,
                     acc_sh, gsem, ssem):
    c = lax.axis_index("c")
    s = lax.axis_index("s")
    w = c * _NS + s

    def zrow(r, carry):
        def zcol(k, carry2):
            gbuf[0, r, pl.ds(k * 16, 16)] = jnp.zeros((16,), jnp.float32)
            return carry2

        lax.fori_loop(0, _D // 16, zcol, 0)
        return carry

    lax.fori_loop(0, _CH, zrow, 0)

    def zacc(k, carry):
        pltpu.sync_copy(gbuf.at[0, pl.ds(0, 64)],
                        acc_sh.at[pl.ds(s * _RPT + k * 64, 64)])
        return carry

    lax.fori_loop(0, _RPT // 64, zacc, 0)

    pltpu.sync_copy(comb_hbm.at[pl.ds(w * _EPW, _EPW)], idx_v)
    plsc.subcore_barrier()

    def unpack(j):
        m = lax.rem(j, 2)

        def body(k, carry):
            base = j * _CH + k * 16

            @pl.when(base < _EPW)
            def _():
                v = idx_v[pl.ds(base, 16)]
                sbuf[m, pl.ds(k * 16, 16)] = lax.bitwise_and(
                    v, jnp.full((16,), 0xFFFF, jnp.int32))
                dbuf[m, pl.ds(k * 16, 16)] = lax.shift_right_logical(v, 16)

            @pl.when(base >= _EPW)
            def _():
                pad = k * 16 + lax.iota(jnp.int32, 16)
                sbuf[m, pl.ds(k * 16, 16)] = pad
                dbuf[m, pl.ds(k * 16, 16)] = _N + pad

            return carry

        lax.fori_loop(0, _CH // 16, body, 0)

    def start_gather(j):
        m = lax.rem(j, 2)
        pltpu.async_copy(hs_hbm.at[sbuf.at[m]], gbuf.at[m], gsem.at[m])

    def wait_gather(j):
        m = lax.rem(j, 2)
        pltpu.make_async_copy(hs_hbm.at[sbuf.at[m]], gbuf.at[m],
                              gsem.at[m]).wait()

    def start_scatter(j):
        m = lax.rem(j, 2)
        pltpu.async_copy(gbuf.at[m], acc_sh.at[dbuf.at[m]], ssem.at[m],
                         add=True)

    def wait_scatter(j):
        m = lax.rem(j, 2)
        pltpu.make_async_copy(gbuf.at[m], acc_sh.at[dbuf.at[m]],
                              ssem.at[m]).wait()

    unpack(0)
    start_gather(0)
    unpack(1)
    start_gather(1)
    wait_gather(0)
    start_scatter(0)

    def chunk(j, carry):
        wait_scatter(j - 1)
        unpack(j + 1)
        start_gather(j + 1)
        wait_gather(j)
        start_scatter(j)
        return carry

    lax.fori_loop(1, _NCHUNK - 1, chunk, 0)
    wait_gather(_NCHUNK - 1)
    start_scatter(_NCHUNK - 1)
    wait_scatter(_NCHUNK - 2)
    wait_scatter(_NCHUNK - 1)
    plsc.subcore_barrier()
    pltpu.sync_copy(acc_sh.at[pl.ds(s * _RPT, _RPT)], out_hbm.at[c, s])


@functools.cache
def _sc_kernels():
    mesh = plsc.VectorSubcoreMesh(**_MESH)
    sc_deg = pl.kernel(
        _sc_deg_body,
        out_type=jax.ShapeDtypeStruct((_NC, _NS, _RPT), jnp.float32),
        mesh=mesh,
        scratch_types=[
            pltpu.VMEM((_EPW,), jnp.int32),
            pltpu.VMEM((1, _CH), jnp.int32),
            pltpu.VMEM((_CH,), jnp.float32),
            pltpu.VMEM((_RPT,), jnp.float32),
            pltpu.VMEM_SHARED((_NP,), jnp.float32),
        ],
    )
    sc_scatter = pl.kernel(
        _sc_scatter_body,
        out_type=jax.ShapeDtypeStruct((_NC, _NS, _RPT, _D), jnp.float32),
        mesh=mesh,
        scratch_types=[
            pltpu.VMEM((_EPW,), jnp.int32),
            pltpu.VMEM((2, _CH), jnp.int32),
            pltpu.VMEM((2, _CH), jnp.int32),
            pltpu.VMEM((2, _CH, _D), jnp.float32),
            pltpu.VMEM_SHARED((_NP, _D), jnp.float32),
            pltpu.SemaphoreType.DMA((2,)),
            pltpu.SemaphoreType.DMA((2,)),
        ],
    )
    return sc_deg, sc_scatter


def _dgn_relu(o, lw, lb, bw, bb):
    logits = jnp.dot(o, lw, preferred_element_type=jnp.float32) + lb[None, :]
    s = jax.nn.softmax(logits, axis=-1)
    cdims = (((0,), (0,)), ((), ()))
    m = lax.dot_general(s, o, cdims,
                        preferred_element_type=jnp.float32) / _N
    q = lax.dot_general(s * s, o * o, cdims,
                        preferred_element_type=jnp.float32) / _N
    var = q - m * m
    wgt = bw * lax.rsqrt(var + _EPS)
    cvec = jnp.sum(m * wgt - bb, axis=0, keepdims=True)
    sw = jnp.dot(s, wgt, preferred_element_type=jnp.float32)
    return jnp.maximum(o + _LAM * (o * sw - cvec), 0.0)


def _tc_prea_body(x_ref, wpre_ref, bpre_ref, w1_ref, hw_ref):
    h = jnp.dot(x_ref[...], wpre_ref[...],
                preferred_element_type=jnp.float32) + bpre_ref[...][None, :]
    hw_ref[...] = jnp.dot(h, w1_ref[...], preferred_element_type=jnp.float32)


def _tc_preb_body(hw_ref, degp_ref, hs_ref, dinv_ref):
    deg = degp_ref[0] + degp_ref[1]
    dinv = lax.rsqrt(deg[: _N] + 1.0)
    dinv_ref[...] = dinv
    hs_ref[...] = dinv * hw_ref[...]


def _tc_mid_body(acc_ref, hs1_ref, dinv_ref, b1_ref, lw_ref, lb_ref, bw_ref,
                 bb_ref, w2_ref, hs2_ref):
    accsum = acc_ref[0, : _N] + acc_ref[1, : _N]
    dinv = dinv_ref[...]
    o = dinv * (accsum + hs1_ref[...]) + b1_ref[...][None, :]
    h = _dgn_relu(o, lw_ref[...], lb_ref[...], bw_ref[...], bb_ref[...])
    hs2_ref[...] = dinv * jnp.dot(h, w2_ref[...],
                                  preferred_element_type=jnp.float32)


def _tc_post_body(acc_ref, hs2_ref, dinv_ref, b2_ref, lw_ref, lb_ref, bw_ref,
                  bb_ref, wjk_ref, bjk_ref, wpost_ref, bpost_ref, out_ref):
    accsum = acc_ref[0, : _N] + acc_ref[1, : _N]
    dinv = dinv_ref[...]
    o = dinv * (accsum + hs2_ref[...]) + b2_ref[...][None, :]
    h = _dgn_relu(o, lw_ref[...], lb_ref[...], bw_ref[...], bb_ref[...])
    t = jnp.dot(h, wjk_ref[...],
                preferred_element_type=jnp.float32) + bjk_ref[...][None, :]
    out_ref[...] = jnp.dot(t, wpost_ref[...],
                           preferred_element_type=jnp.float32) \
        + bpost_ref[...][None, :]


_tc_prea = pl.pallas_call(
    _tc_prea_body,
    out_shape=jax.ShapeDtypeStruct((_N, _D), jnp.float32),
)

_tc_preb = pl.pallas_call(
    _tc_preb_body,
    out_shape=[jax.ShapeDtypeStruct((_N, _D), jnp.float32),
               jax.ShapeDtypeStruct((_N, 1), jnp.float32)],
)

_tc_mid = pl.pallas_call(
    _tc_mid_body,
    out_shape=jax.ShapeDtypeStruct((_N, _D), jnp.float32),
)

_tc_post = pl.pallas_call(
    _tc_post_body,
    out_shape=jax.ShapeDtypeStruct((_N, _D), jnp.float32),
)


def _pad_group_params(lw, lb, bw, bb):
    lw_p = jnp.pad(lw, ((0, 0), (0, _GP - _G)))
    lb_p = jnp.pad(lb, (0, _GP - _G), constant_values=-1e30)
    bw_p = jnp.pad(bw.reshape(_G, _D), ((0, _GP - _G), (0, 0)))
    bb_p = jnp.pad(bb.reshape(_G, _D), ((0, _GP - _G), (0, 0)))
    return lw_p, lb_p, bw_p, bb_p


def kernel(x, adj, W_pre, b_pre, W1, b1, dgn1_lw, dgn1_lb, dgn1_bw, dgn1_bb,
           W2, b2, dgn2_lw, dgn2_lb, dgn2_bw, dgn2_bb, W_jk, b_jk, W_post,
           b_post):
    src, dst = adj[0], adj[1]
    comb = src + dst * 65536

    lw1, lb1, bw1, bb1 = _pad_group_params(dgn1_lw, dgn1_lb, dgn1_bw, dgn1_bb)
    lw2, lb2, bw2, bb2 = _pad_group_params(dgn2_lw, dgn2_lb, dgn2_bw, dgn2_bb)

    sc_deg, sc_scatter = _sc_kernels()
    degp = sc_deg(comb).reshape(_NC, _NP, 1)
    hw1 = _tc_prea(x, W_pre, b_pre, W1)
    hs1, dinv = _tc_preb(hw1, degp)
    acc1 = sc_scatter(hs1, comb).reshape(_NC, _NP, _D)
    hs2 = _tc_mid(acc1, hs1, dinv, b1, lw1, lb1, bw1, bb1, W2)
    acc2 = sc_scatter(hs2, comb).reshape(_NC, _NP, _D)
    out = _tc_post(acc2, hs2, dinv, b2, lw2, lb2, bw2, bb2, W_jk, b_jk,
                   W_post, b_post)
    return out


# speedup vs baseline: 1.0553x; 1.0547x over previous
import functools

import jax
import jax.numpy as jnp
from jax import lax
from jax.experimental import pallas as pl
from jax.experimental.pallas import tpu as pltpu
from jax.experimental.pallas import tpu_sc as plsc

_N = 10000
_E = 320000
_D = 128
_G = 5
_GP = 8
_LAM = 0.01
_EPS = 1e-5

_NC = 2
_NS = 16
_NW = _NC * _NS
_EPW = _E // _NW
_CH = 128
_NCHUNK = -(-_EPW // _CH)
_NP = 10240
_RPT = _NP // _NS

_MESH = dict(core_axis_name="c", subcore_axis_name="s", num_cores=_NC,
             num_subcores=_NS)


def _sc_deg_body(dst_hbm, out_hbm, idx_v, dbuf, ones_v, z_v, hist_sh):
    c = lax.axis_index("c")
    s = lax.axis_index("s")
    w = c * _NS + s

    def fill_ones(i, carry):
        ones_v[pl.ds(i * 16, 16)] = jnp.full((16,), 1.0, jnp.float32)
        return carry

    lax.fori_loop(0, _CH // 16, fill_ones, 0)

    def fill_z(i, carry):
        z_v[pl.ds(i * 16, 16)] = jnp.zeros((16,), jnp.float32)
        return carry

    lax.fori_loop(0, _RPT // 16, fill_z, 0)

    pltpu.sync_copy(dst_hbm.at[pl.ds(w * _EPW, _EPW)], idx_v)
    pltpu.sync_copy(z_v, hist_sh.at[pl.ds(s * _RPT, _RPT)])
    plsc.subcore_barrier()

    def chunk(j, carry):
        def unpack(k, carry2):
            base = j * _CH + k * 16

            @pl.when(base < _EPW)
            def _():
                dbuf[0, pl.ds(k * 16, 16)] = idx_v[pl.ds(base, 16)]

            @pl.when(base >= _EPW)
            def _():
                dbuf[0, pl.ds(k * 16, 16)] = (
                    _N + k * 16 + lax.iota(jnp.int32, 16))

            return carry2

        lax.fori_loop(0, _CH // 16, unpack, 0)
        pltpu.sync_copy(ones_v, hist_sh.at[dbuf.at[0]], add=True)
        return carry

    lax.fori_loop(0, _NCHUNK, chunk, 0)
    plsc.subcore_barrier()
    pltpu.sync_copy(hist_sh.at[pl.ds(s * _RPT, _RPT)], out_hbm.at[c, s])


def _sc_scatter_body(hs_hbm, comb_hbm, out_hbm, idx_v, sbuf, dbuf, gbuf,
                     acc_sh, gsem, ssem):
    c = lax.axis_index("c")
    s = lax.axis_index("s")
    w = c * _NS + s

    def zrow(r, carry):
        def zcol(k, carry2):
            gbuf[0, r, pl.ds(k * 16, 16)] = jnp.zeros((16,), jnp.float32)
            return carry2

        lax.fori_loop(0, _D // 16, zcol, 0)
        return carry

    lax.fori_loop(0, _CH, zrow, 0)

    def zacc(k, carry):
        pltpu.sync_copy(gbuf.at[0, pl.ds(0, 64)],
                        acc_sh.at[pl.ds(s * _RPT + k * 64, 64)])
        return carry

    lax.fori_loop(0, _RPT // 64, zacc, 0)

    pltpu.sync_copy(comb_hbm.at[pl.ds(w * _EPW, _EPW)], idx_v)
    plsc.subcore_barrier()

    def unpack(j):
        m = lax.rem(j, 2)

        def body(k, carry):
            base = j * _CH + k * 16

            @pl.when(base < _EPW)
            def _():
                v = idx_v[pl.ds(base, 16)]
                sbuf[m, pl.ds(k * 16, 16)] = lax.bitwise_and(
                    v, jnp.full((16,), 0xFFFF, jnp.int32))
                dbuf[m, pl.ds(k * 16, 16)] = lax.shift_right_logical(v, 16)

            @pl.when(base >= _EPW)
            def _():
                pad = k * 16 + lax.iota(jnp.int32, 16)
                sbuf[m, pl.ds(k * 16, 16)] = pad
                dbuf[m, pl.ds(k * 16, 16)] = _N + pad

            return carry

        lax.fori_loop(0, _CH // 16, body, 0)

    def start_gather(j):
        m = lax.rem(j, 2)
        pltpu.async_copy(hs_hbm.at[sbuf.at[m]], gbuf.at[m], gsem.at[m])

    def wait_gather(j):
        m = lax.rem(j, 2)
        pltpu.make_async_copy(hs_hbm.at[sbuf.at[m]], gbuf.at[m],
                              gsem.at[m]).wait()

    def start_scatter(j):
        m = lax.rem(j, 2)
        pltpu.async_copy(gbuf.at[m], acc_sh.at[dbuf.at[m]], ssem.at[m],
                         add=True)

    def wait_scatter(j):
        m = lax.rem(j, 2)
        pltpu.make_async_copy(gbuf.at[m], acc_sh.at[dbuf.at[m]],
                              ssem.at[m]).wait()

    unpack(0)
    start_gather(0)
    unpack(1)
    start_gather(1)
    wait_gather(0)
    start_scatter(0)

    def chunk(j, carry):
        wait_scatter(j - 1)
        unpack(j + 1)
        start_gather(j + 1)
        wait_gather(j)
        start_scatter(j)
        return carry

    lax.fori_loop(1, _NCHUNK - 1, chunk, 0)
    wait_gather(_NCHUNK - 1)
    start_scatter(_NCHUNK - 1)
    wait_scatter(_NCHUNK - 2)
    wait_scatter(_NCHUNK - 1)
    plsc.subcore_barrier()
    pltpu.sync_copy(acc_sh.at[pl.ds(s * _RPT, _RPT)], out_hbm.at[c, s])


@functools.cache
def _sc_kernels():
    mesh = plsc.VectorSubcoreMesh(**_MESH)
    sc_deg = pl.kernel(
        _sc_deg_body,
        out_type=jax.ShapeDtypeStruct((_NC, _NS, _RPT), jnp.float32),
        mesh=mesh,
        scratch_types=[
            pltpu.VMEM((_EPW,), jnp.int32),
            pltpu.VMEM((1, _CH), jnp.int32),
            pltpu.VMEM((_CH,), jnp.float32),
            pltpu.VMEM((_RPT,), jnp.float32),
            pltpu.VMEM_SHARED((_NP,), jnp.float32),
        ],
    )
    sc_scatter = pl.kernel(
        _sc_scatter_body,
        out_type=jax.ShapeDtypeStruct((_NC, _NS, _RPT, _D), jnp.float32),
        mesh=mesh,
        scratch_types=[
            pltpu.VMEM((_EPW,), jnp.int32),
            pltpu.VMEM((2, _CH), jnp.int32),
            pltpu.VMEM((2, _CH), jnp.int32),
            pltpu.VMEM((2, _CH, _D), jnp.float32),
            pltpu.VMEM_SHARED((_NP, _D), jnp.float32),
            pltpu.SemaphoreType.DMA((2,)),
            pltpu.SemaphoreType.DMA((2,)),
        ],
    )
    return sc_deg, sc_scatter


_NB = _NP // _D


def _mask80():
    n = (lax.broadcasted_iota(jnp.int32, (_NB, _D), 0) * _D
         + lax.broadcasted_iota(jnp.int32, (_NB, _D), 1))
    return jnp.where(n < _N, 1.0, 0.0).astype(jnp.float32)


def _rowmul(m, col80):
    return (m.reshape(_NB, _D, _D) * col80[:, :, None]).reshape(_NP, _D)


def _dgn_relu(o, lw, lb, bw, bb):
    logits = jnp.dot(o, lw, preferred_element_type=jnp.float32) + lb[None, :]
    s = jax.nn.softmax(logits, axis=-1)
    cdims = (((0,), (0,)), ((), ()))
    m = lax.dot_general(s, o, cdims,
                        preferred_element_type=jnp.float32) / _N
    q = lax.dot_general(s * s, o * o, cdims,
                        preferred_element_type=jnp.float32) / _N
    var = q - m * m
    wgt = bw * lax.rsqrt(var + _EPS)
    cvec = jnp.sum(m * wgt - bb, axis=0, keepdims=True)
    sw = jnp.dot(s, wgt, preferred_element_type=jnp.float32)
    return jnp.maximum(o + _LAM * (o * sw - cvec), 0.0)


def _conv_out(acc_ref, hs_ref, dinvm, b):
    accsum = acc_ref[0] + acc_ref[1] + hs_ref[...]
    o3 = (accsum.reshape(_NB, _D, _D) * dinvm[:, :, None]
          + _mask80()[:, :, None] * b[None, None, :])
    return o3.reshape(_NP, _D)


def _tc_prea_body(x_ref, wpre_ref, bpre_ref, w1_ref, hw_ref):
    h = jnp.dot(x_ref[...], wpre_ref[...],
                preferred_element_type=jnp.float32) + bpre_ref[...][None, :]
    hw_ref[: _N] = jnp.dot(h, w1_ref[...], preferred_element_type=jnp.float32)
    hw_ref[_N:] = jnp.zeros((_NP - _N, _D), jnp.float32)


def _tc_preb_body(hw_ref, degp_ref, hs_ref, dinv_ref):
    deg = degp_ref[0] + degp_ref[1]
    dinvm = lax.rsqrt(deg + 1.0) * _mask80()
    dinv_ref[...] = dinvm
    hs_ref[...] = _rowmul(hw_ref[...], dinvm)


def _tc_mid_body(acc_ref, hs1_ref, dinv_ref, b1_ref, lw_ref, lb_ref, bw_ref,
                 bb_ref, w2_ref, hs2_ref):
    dinvm = dinv_ref[...]
    o = _conv_out(acc_ref, hs1_ref, dinvm, b1_ref[...])
    h = _dgn_relu(o, lw_ref[...], lb_ref[...], bw_ref[...], bb_ref[...])
    hs2_ref[...] = _rowmul(
        jnp.dot(h, w2_ref[...], preferred_element_type=jnp.float32), dinvm)


def _tc_post_body(acc_ref, hs2_ref, dinv_ref, b2_ref, lw_ref, lb_ref, bw_ref,
                  bb_ref, wjk_ref, bjk_ref, wpost_ref, bpost_ref, out_ref):
    o = _conv_out(acc_ref, hs2_ref, dinv_ref[...], b2_ref[...])
    h = _dgn_relu(o, lw_ref[...], lb_ref[...], bw_ref[...], bb_ref[...])
    t = jnp.dot(h, wjk_ref[...],
                preferred_element_type=jnp.float32) + bjk_ref[...][None, :]
    out_ref[...] = (jnp.dot(t, wpost_ref[...],
                            preferred_element_type=jnp.float32)
                    + bpost_ref[...][None, :])[: _N]


_tc_prea = pl.pallas_call(
    _tc_prea_body,
    out_shape=jax.ShapeDtypeStruct((_NP, _D), jnp.float32),
)

_tc_preb = pl.pallas_call(
    _tc_preb_body,
    out_shape=[jax.ShapeDtypeStruct((_NP, _D), jnp.float32),
               jax.ShapeDtypeStruct((_NB, _D), jnp.float32)],
)

_tc_mid = pl.pallas_call(
    _tc_mid_body,
    out_shape=jax.ShapeDtypeStruct((_NP, _D), jnp.float32),
)

_tc_post = pl.pallas_call(
    _tc_post_body,
    out_shape=jax.ShapeDtypeStruct((_N, _D), jnp.float32),
)


def _pad_group_params(lw, lb, bw, bb):
    lw_p = jnp.pad(lw, ((0, 0), (0, _GP - _G)))
    lb_p = jnp.pad(lb, (0, _GP - _G), constant_values=-1e30)
    bw_p = jnp.pad(bw.reshape(_G, _D), ((0, _GP - _G), (0, 0)))
    bb_p = jnp.pad(bb.reshape(_G, _D), ((0, _GP - _G), (0, 0)))
    return lw_p, lb_p, bw_p, bb_p


def kernel(x, adj, W_pre, b_pre, W1, b1, dgn1_lw, dgn1_lb, dgn1_bw, dgn1_bb,
           W2, b2, dgn2_lw, dgn2_lb, dgn2_bw, dgn2_bb, W_jk, b_jk, W_post,
           b_post):
    src, dst = adj[0], adj[1]
    comb = src + dst * 65536

    lw1, lb1, bw1, bb1 = _pad_group_params(dgn1_lw, dgn1_lb, dgn1_bw, dgn1_bb)
    lw2, lb2, bw2, bb2 = _pad_group_params(dgn2_lw, dgn2_lb, dgn2_bw, dgn2_bb)

    sc_deg, sc_scatter = _sc_kernels()
    degp = sc_deg(dst).reshape(_NC, _NB, _D)
    hw1 = _tc_prea(x, W_pre, b_pre, W1)
    hs1, dinv = _tc_preb(hw1, degp)
    acc1 = sc_scatter(hs1, comb).reshape(_NC, _NP, _D)
    hs2 = _tc_mid(acc1, hs1, dinv, b1, lw1, lb1, bw1, bb1, W2)
    acc2 = sc_scatter(hs2, comb).reshape(_NC, _NP, _D)
    out = _tc_post(acc2, hs2, dinv, b2, lw2, lb2, bw2, bb2, W_jk, b_jk,
                   W_post, b_post)
    return out
